# baseline (speedup 1.0000x reference)
import jax
import jax.numpy as jnp
from jax import lax
from jax.experimental import pallas as pl
from jax.experimental.pallas import tpu as pltpu
from jax.experimental.pallas import tpu_sc as plsc

N_NODES = 10000
N_EDGES = 320000
D_IN = 128
D_HID = 128
D_HID2 = 64
DH = 64

NC = 2
NS = 16
NW = NC * NS

EPW = N_EDGES // NW
CH = 80
NCHUNK = EPW // CH
RPS = N_NODES // NS
ZR = 125


def _zero_fill(zbuf, rows, width):
  z16 = jnp.zeros((16,), jnp.float32)

  def body(i, carry):
    for j in range(width // 16):
      zbuf[i, pl.ds(j * 16, 16)] = z16
    return carry

  lax.fori_loop(0, rows, body, 0)


def _writeback(acc_sh, out_hbm, c, s):
  @pl.when(s < 10)
  def _():
    pltpu.sync_copy(acc_sh.at[pl.ds(s * 1000, 1000)],
                    out_hbm.at[c, pl.ds(s * 1000, 1000)])


def _deg_body(dst_hbm, zeros_hbm, ones_hbm, out_hbm, dbuf, ones_v, acc_sh):
  c = lax.axis_index("c")
  s = lax.axis_index("s")
  wid = s * NC + c

  pltpu.sync_copy(ones_hbm, ones_v)
  @pl.when(s < 10)
  def _():
    pltpu.sync_copy(zeros_hbm.at[pl.ds(s * 1000, 1000)],
                    acc_sh.at[pl.ds(s * 1000, 1000)])
  plsc.subcore_barrier()

  pltpu.sync_copy(dst_hbm.at[wid], dbuf)

  def chunk(i, carry):
    pltpu.sync_copy(ones_v, acc_sh.at[dbuf.at[i]], add=True)
    return carry

  lax.fori_loop(0, NCHUNK, chunk, 0)
  plsc.subcore_barrier()
  _writeback(acc_sh, out_hbm, c, s)


def _deg_kernel():
  mesh = plsc.VectorSubcoreMesh(core_axis_name="c", subcore_axis_name="s")
  return pl.kernel(
      _deg_body,
      compiler_params=pltpu.CompilerParams(use_tc_tiling_on_sc=False),
      out_type=jax.ShapeDtypeStruct((NC, N_NODES, 1), jnp.float32),
      mesh=mesh,
      scratch_types=[
          pltpu.VMEM((NCHUNK, CH), jnp.int32),
          pltpu.VMEM((CH, 1), jnp.float32),
          pltpu.VMEM_SHARED((N_NODES, 1), jnp.float32),
      ],
  )


def _agg_pass(h_hbm, out_hbm, sbuf, dbuf, rows_a, rows_b, zbuf, acc_sh,
              gsa, gsb, ssa, ssb, c, s):
  for k in range(RPS // ZR):
    pltpu.sync_copy(zbuf, acc_sh.at[pl.ds(s * RPS + k * ZR, ZR)])
  plsc.subcore_barrier()

  def g_start(buf, gsem, i):
    pltpu.async_copy(h_hbm.at[sbuf.at[i]], buf, gsem)

  def g_wait(gsem):
    pltpu.make_async_copy(h_hbm.at[pl.ds(0, CH)], rows_a, gsem).wait()

  def s_start(buf, ssem, i):
    pltpu.async_copy(buf, acc_sh.at[dbuf.at[i]], ssem, add=True)

  def s_wait(ssem):
    pltpu.make_async_copy(rows_a, acc_sh.at[pl.ds(0, CH)], ssem).wait()

  g_start(rows_a, gsa, 0)
  pltpu.async_copy(zbuf.at[pl.ds(0, CH)], acc_sh.at[dbuf.at[0]], ssb,
                   add=True)

  def body(k, carry):
    i = 2 * k
    g_wait(gsa)
    s_start(rows_a, ssa, i)
    s_wait(ssb)
    g_start(rows_b, gsb, i + 1)
    g_wait(gsb)
    s_start(rows_b, ssb, i + 1)
    s_wait(ssa)
    g_start(rows_a, gsa, i + 2)
    return carry

  lax.fori_loop(0, (NCHUNK - 1) // 2, body, 0)
  g_wait(gsa)
  s_start(rows_a, ssa, NCHUNK - 1)
  s_wait(ssb)
  s_wait(ssa)
  plsc.subcore_barrier()
  _writeback(acc_sh, out_hbm, c, s)
  plsc.subcore_barrier()


def _make_agg_kernel(nhalves):
  def body(*refs):
    h_hbms = refs[:nhalves]
    src_hbm = refs[nhalves]
    dst_hbm = refs[nhalves + 1]
    out_hbms = refs[nhalves + 2:2 * nhalves + 2]
    (sbuf, dbuf, rows_a, rows_b, zbuf, acc_sh,
     gsa, gsb, ssa, ssb) = refs[2 * nhalves + 2:]

    c = lax.axis_index("c")
    s = lax.axis_index("s")
    wid = s * NC + c

    _zero_fill(zbuf, ZR, DH)
    pltpu.sync_copy(src_hbm.at[wid], sbuf)
    pltpu.sync_copy(dst_hbm.at[wid], dbuf)

    for h_hbm, out_hbm in zip(h_hbms, out_hbms):
      _agg_pass(h_hbm, out_hbm, sbuf, dbuf, rows_a, rows_b, zbuf, acc_sh,
                gsa, gsb, ssa, ssb, c, s)

  mesh = plsc.VectorSubcoreMesh(core_axis_name="c", subcore_axis_name="s")
  return pl.kernel(
      body,
      compiler_params=pltpu.CompilerParams(use_tc_tiling_on_sc=False),
      out_type=[jax.ShapeDtypeStruct((NC, N_NODES, DH), jnp.float32)
                for _ in range(nhalves)],
      mesh=mesh,
      scratch_types=[
          pltpu.VMEM((NCHUNK, CH), jnp.int32),
          pltpu.VMEM((NCHUNK, CH), jnp.int32),
          pltpu.VMEM((CH, DH), jnp.float32),
          pltpu.VMEM((CH, DH), jnp.float32),
          pltpu.VMEM((ZR, DH), jnp.float32),
          pltpu.VMEM_SHARED((N_NODES, DH), jnp.float32),
          pltpu.SemaphoreType.DMA,
          pltpu.SemaphoreType.DMA,
          pltpu.SemaphoreType.DMA,
          pltpu.SemaphoreType.DMA,
      ],
  )


BM = 1000


def _tc1_body(x_ref, w_ref, da_ref, db_ref, hlo_ref, hhi_ref, dinv_ref):
  deg = da_ref[...] + db_ref[...] + 1.0
  dinv = lax.rsqrt(deg)
  h = dinv * jnp.dot(x_ref[...], w_ref[...],
                     preferred_element_type=jnp.float32)
  hlo_ref[...] = h[:, :DH]
  hhi_ref[...] = h[:, DH:]
  dinv_ref[...] = dinv


def _tc2_body(alo_ref, ahi_ref, hlo_ref, hhi_ref, dinv_ref, b1_ref, w2_ref,
              out_ref):
  dinv = dinv_ref[...]
  zlo = (alo_ref[0] + alo_ref[1] + hlo_ref[...]) * dinv + b1_ref[:, :DH]
  zhi = (ahi_ref[0] + ahi_ref[1] + hhi_ref[...]) * dinv + b1_ref[:, DH:]
  r = jnp.concatenate([jnp.maximum(zlo, 0.0), jnp.maximum(zhi, 0.0)], axis=1)
  out_ref[...] = dinv * jnp.dot(r, w2_ref[...],
                                preferred_element_type=jnp.float32)


def _tc3_body(agg_ref, h2p_ref, dinv_ref, b2_ref, wfct_ref, bfc_ref, out_ref):
  dinv = dinv_ref[...]
  z = (agg_ref[0] + agg_ref[1] + h2p_ref[...]) * dinv + b2_ref[...]
  r = jnp.maximum(z, 0.0)
  l0 = jnp.sum(r * wfct_ref[0:1, :], axis=1, keepdims=True) + bfc_ref[:, 0:1]
  l1 = jnp.sum(r * wfct_ref[1:2, :], axis=1, keepdims=True) + bfc_ref[:, 1:2]
  m = jnp.maximum(l0, l1)
  lse = m + jnp.log(jnp.exp(l0 - m) + jnp.exp(l1 - m))
  out_ref[...] = jnp.concatenate([l0 - lse, l1 - lse], axis=1)


def _tc1(x, w1, dega, degb):
  grid = (N_NODES // BM,)
  return pl.pallas_call(
      _tc1_body,
      grid=grid,
      in_specs=[
          pl.BlockSpec((BM, D_IN), lambda i: (i, 0)),
          pl.BlockSpec((D_IN, D_HID), lambda i: (0, 0)),
          pl.BlockSpec((BM, 1), lambda i: (i, 0)),
          pl.BlockSpec((BM, 1), lambda i: (i, 0)),
      ],
      out_specs=[
          pl.BlockSpec((BM, DH), lambda i: (i, 0)),
          pl.BlockSpec((BM, DH), lambda i: (i, 0)),
          pl.BlockSpec((BM, 1), lambda i: (i, 0)),
      ],
      out_shape=[
          jax.ShapeDtypeStruct((N_NODES, DH), jnp.float32),
          jax.ShapeDtypeStruct((N_NODES, DH), jnp.float32),
          jax.ShapeDtypeStruct((N_NODES, 1), jnp.float32),
      ],
  )(x, w1, dega, degb)


def _tc2(alo, ahi, hlo, hhi, dinv, b1, w2):
  grid = (N_NODES // BM,)
  return pl.pallas_call(
      _tc2_body,
      grid=grid,
      in_specs=[
          pl.BlockSpec((NC, BM, DH), lambda i: (0, i, 0)),
          pl.BlockSpec((NC, BM, DH), lambda i: (0, i, 0)),
          pl.BlockSpec((BM, DH), lambda i: (i, 0)),
          pl.BlockSpec((BM, DH), lambda i: (i, 0)),
          pl.BlockSpec((BM, 1), lambda i: (i, 0)),
          pl.BlockSpec((1, D_HID), lambda i: (0, 0)),
          pl.BlockSpec((D_HID, D_HID2), lambda i: (0, 0)),
      ],
      out_specs=pl.BlockSpec((BM, D_HID2), lambda i: (i, 0)),
      out_shape=jax.ShapeDtypeStruct((N_NODES, D_HID2), jnp.float32),
  )(alo, ahi, hlo, hhi, dinv, b1, w2)


def _tc3(agg, h2p, dinv, b2, wfct, bfc2):
  grid = (N_NODES // BM,)
  return pl.pallas_call(
      _tc3_body,
      grid=grid,
      in_specs=[
          pl.BlockSpec((NC, BM, D_HID2), lambda i: (0, i, 0)),
          pl.BlockSpec((BM, D_HID2), lambda i: (i, 0)),
          pl.BlockSpec((BM, 1), lambda i: (i, 0)),
          pl.BlockSpec((1, D_HID2), lambda i: (0, 0)),
          pl.BlockSpec((2, D_HID2), lambda i: (0, 0)),
          pl.BlockSpec((1, 2), lambda i: (0, 0)),
      ],
      out_specs=pl.BlockSpec((BM, 2), lambda i: (i, 0)),
      out_shape=jax.ShapeDtypeStruct((N_NODES, 2), jnp.float32),
  )(agg, h2p, dinv, b2, wfct, bfc2)


def kernel(x, edge_index, W1, b1, W2, b2, Wfc, bfc):
  ei = edge_index.astype(jnp.int32)
  src = ei[0].reshape(NW, NCHUNK, CH)
  dst = ei[1].reshape(NW, NCHUNK, CH)

  zeros_n1 = jnp.zeros((N_NODES, 1), jnp.float32)
  ones_ch1 = jnp.ones((CH, 1), jnp.float32)
  deg2 = _deg_kernel()(dst, zeros_n1, ones_ch1)

  hlo, hhi, dinv = _tc1(x, W1, deg2[0], deg2[1])
  alo, ahi = _make_agg_kernel(2)(hlo, hhi, src, dst)
  h2p = _tc2(alo, ahi, hlo, hhi, dinv, b1.reshape(1, D_HID), W2)
  (agg2,) = _make_agg_kernel(1)(h2p, src, dst)
  out = _tc3(agg2, h2p, dinv, b2.reshape(1, D_HID2), Wfc.T,
             bfc.reshape(1, 2))
  return out


# speedup vs baseline: 1.4122x; 1.4122x over previous
import jax
import jax.numpy as jnp
from jax import lax
from jax.experimental import pallas as pl
from jax.experimental.pallas import tpu as pltpu
from jax.experimental.pallas import tpu_sc as plsc

N_NODES = 10000
N_EDGES = 320000
D_IN = 128
D_HID = 128
D_HID2 = 64
DH = 64

NC = 2
NS = 16
NW = NC * NS

EPW = N_EDGES // NW
CH = 80
NCHUNK = EPW // CH
RPS = N_NODES // NS
ZR = 125


def _zero_fill(zbuf, rows, width):
  z16 = jnp.zeros((16,), jnp.float32)

  def body(i, carry):
    for j in range(width // 16):
      zbuf[i, pl.ds(j * 16, 16)] = z16
    return carry

  lax.fori_loop(0, rows, body, 0)


def _writeback(acc_sh, out_hbm, c, s):
  @pl.when(s < 10)
  def _():
    pltpu.sync_copy(acc_sh.at[pl.ds(s * 1000, 1000)],
                    out_hbm.at[c, pl.ds(s * 1000, 1000)])


def _deg_body(dst_hbm, zeros_hbm, ones_hbm, out_hbm, dbuf, ones_v, acc_sh):
  c = lax.axis_index("c")
  s = lax.axis_index("s")
  wid = s * NC + c

  pltpu.sync_copy(ones_hbm, ones_v)
  @pl.when(s < 10)
  def _():
    pltpu.sync_copy(zeros_hbm.at[pl.ds(s * 1000, 1000)],
                    acc_sh.at[pl.ds(s * 1000, 1000)])
  plsc.subcore_barrier()

  pltpu.sync_copy(dst_hbm.at[wid], dbuf)

  def chunk(i, carry):
    pltpu.sync_copy(ones_v, acc_sh.at[dbuf.at[i]], add=True)
    return carry

  lax.fori_loop(0, NCHUNK, chunk, 0)
  plsc.subcore_barrier()
  _writeback(acc_sh, out_hbm, c, s)


def _deg_kernel():
  mesh = plsc.VectorSubcoreMesh(core_axis_name="c", subcore_axis_name="s")
  return pl.kernel(
      _deg_body,
      compiler_params=pltpu.CompilerParams(use_tc_tiling_on_sc=False),
      out_type=jax.ShapeDtypeStruct((NC, N_NODES, 1), jnp.float32),
      mesh=mesh,
      scratch_types=[
          pltpu.VMEM((NCHUNK, CH), jnp.int32),
          pltpu.VMEM((CH, 1), jnp.float32),
          pltpu.VMEM_SHARED((N_NODES, 1), jnp.float32),
      ],
  )


def _agg_pass(h_hbm, out_hbm, sbuf, dbuf, rows, zbuf, acc_sh, gs, ss, c, s):
  for k in range(RPS // ZR):
    pltpu.sync_copy(zbuf, acc_sh.at[pl.ds(s * RPS + k * ZR, ZR)])
  plsc.subcore_barrier()

  def g_start(r, i):
    pltpu.async_copy(h_hbm.at[sbuf.at[i]], rows[r], gs[r])

  def g_wait(r):
    pltpu.make_async_copy(h_hbm.at[pl.ds(0, CH)], rows[r], gs[r]).wait()

  def s_start(r, i):
    pltpu.async_copy(rows[r], acc_sh.at[dbuf.at[i]], ss[r], add=True)

  def s_wait(r):
    pltpu.make_async_copy(rows[r], acc_sh.at[pl.ds(0, CH)], ss[r]).wait()

  def step(r, i):
    g_wait(r)
    s_start(r, i)
    s_wait((r + 2) % 3)
    g_start((r + 2) % 3, i + 2)

  g_start(0, 0)
  g_start(1, 1)
  pltpu.async_copy(zbuf.at[pl.ds(0, CH)], acc_sh.at[dbuf.at[0]], ss[2],
                   add=True)

  def body(k, carry):
    i = 3 * k
    step(0, i)
    step(1, i + 1)
    step(2, i + 2)
    return carry

  lax.fori_loop(0, (NCHUNK - 2) // 3, body, 0)
  g_wait(0)
  s_start(0, NCHUNK - 2)
  s_wait(2)
  g_wait(1)
  s_start(1, NCHUNK - 1)
  s_wait(0)
  s_wait(1)
  plsc.subcore_barrier()
  _writeback(acc_sh, out_hbm, c, s)
  plsc.subcore_barrier()


def _make_agg_kernel(nhalves):
  def body(*refs):
    h_hbms = refs[:nhalves]
    src_hbm = refs[nhalves]
    dst_hbm = refs[nhalves + 1]
    out_hbms = refs[nhalves + 2:2 * nhalves + 2]
    (sbuf, dbuf, rows_0, rows_1, rows_2, zbuf, acc_sh,
     gs0, gs1, gs2, ss0, ss1, ss2) = refs[2 * nhalves + 2:]

    c = lax.axis_index("c")
    s = lax.axis_index("s")
    wid = s * NC + c

    _zero_fill(zbuf, ZR, DH)
    pltpu.sync_copy(src_hbm.at[wid], sbuf)
    pltpu.sync_copy(dst_hbm.at[wid], dbuf)

    for h_hbm, out_hbm in zip(h_hbms, out_hbms):
      _agg_pass(h_hbm, out_hbm, sbuf, dbuf, (rows_0, rows_1, rows_2), zbuf,
                acc_sh, (gs0, gs1, gs2), (ss0, ss1, ss2), c, s)

  mesh = plsc.VectorSubcoreMesh(core_axis_name="c", subcore_axis_name="s")
  return pl.kernel(
      body,
      compiler_params=pltpu.CompilerParams(use_tc_tiling_on_sc=False),
      out_type=[jax.ShapeDtypeStruct((NC, N_NODES, DH), jnp.float32)
                for _ in range(nhalves)],
      mesh=mesh,
      scratch_types=[
          pltpu.VMEM((NCHUNK, CH), jnp.int32),
          pltpu.VMEM((NCHUNK, CH), jnp.int32),
          pltpu.VMEM((CH, DH), jnp.float32),
          pltpu.VMEM((CH, DH), jnp.float32),
          pltpu.VMEM((CH, DH), jnp.float32),
          pltpu.VMEM((ZR, DH), jnp.float32),
          pltpu.VMEM_SHARED((N_NODES, DH), jnp.float32),
          pltpu.SemaphoreType.DMA,
          pltpu.SemaphoreType.DMA,
          pltpu.SemaphoreType.DMA,
          pltpu.SemaphoreType.DMA,
          pltpu.SemaphoreType.DMA,
          pltpu.SemaphoreType.DMA,
      ],
  )


BM = 1000


def _tc1_body(x_ref, w_ref, da_ref, db_ref, hlo_ref, hhi_ref, dinv_ref):
  deg = da_ref[...] + db_ref[...] + 1.0
  dinv = lax.rsqrt(deg)
  h = dinv * jnp.dot(x_ref[...], w_ref[...],
                     preferred_element_type=jnp.float32)
  hlo_ref[...] = h[:, :DH]
  hhi_ref[...] = h[:, DH:]
  dinv_ref[...] = dinv


def _tc2_body(alo_ref, ahi_ref, hlo_ref, hhi_ref, dinv_ref, b1_ref, w2_ref,
              out_ref):
  dinv = dinv_ref[...]
  zlo = (alo_ref[0] + alo_ref[1] + hlo_ref[...]) * dinv + b1_ref[:, :DH]
  zhi = (ahi_ref[0] + ahi_ref[1] + hhi_ref[...]) * dinv + b1_ref[:, DH:]
  r = jnp.concatenate([jnp.maximum(zlo, 0.0), jnp.maximum(zhi, 0.0)], axis=1)
  out_ref[...] = dinv * jnp.dot(r, w2_ref[...],
                                preferred_element_type=jnp.float32)


def _tc3_body(agg_ref, h2p_ref, dinv_ref, b2_ref, wfct_ref, bfc_ref, out_ref):
  dinv = dinv_ref[...]
  z = (agg_ref[0] + agg_ref[1] + h2p_ref[...]) * dinv + b2_ref[...]
  r = jnp.maximum(z, 0.0)
  l0 = jnp.sum(r * wfct_ref[0:1, :], axis=1, keepdims=True) + bfc_ref[:, 0:1]
  l1 = jnp.sum(r * wfct_ref[1:2, :], axis=1, keepdims=True) + bfc_ref[:, 1:2]
  m = jnp.maximum(l0, l1)
  lse = m + jnp.log(jnp.exp(l0 - m) + jnp.exp(l1 - m))
  out_ref[...] = jnp.concatenate([l0 - lse, l1 - lse], axis=1)


def _tc1(x, w1, dega, degb):
  grid = (N_NODES // BM,)
  return pl.pallas_call(
      _tc1_body,
      grid=grid,
      in_specs=[
          pl.BlockSpec((BM, D_IN), lambda i: (i, 0)),
          pl.BlockSpec((D_IN, D_HID), lambda i: (0, 0)),
          pl.BlockSpec((BM, 1), lambda i: (i, 0)),
          pl.BlockSpec((BM, 1), lambda i: (i, 0)),
      ],
      out_specs=[
          pl.BlockSpec((BM, DH), lambda i: (i, 0)),
          pl.BlockSpec((BM, DH), lambda i: (i, 0)),
          pl.BlockSpec((BM, 1), lambda i: (i, 0)),
      ],
      out_shape=[
          jax.ShapeDtypeStruct((N_NODES, DH), jnp.float32),
          jax.ShapeDtypeStruct((N_NODES, DH), jnp.float32),
          jax.ShapeDtypeStruct((N_NODES, 1), jnp.float32),
      ],
  )(x, w1, dega, degb)


def _tc2(alo, ahi, hlo, hhi, dinv, b1, w2):
  grid = (N_NODES // BM,)
  return pl.pallas_call(
      _tc2_body,
      grid=grid,
      in_specs=[
          pl.BlockSpec((NC, BM, DH), lambda i: (0, i, 0)),
          pl.BlockSpec((NC, BM, DH), lambda i: (0, i, 0)),
          pl.BlockSpec((BM, DH), lambda i: (i, 0)),
          pl.BlockSpec((BM, DH), lambda i: (i, 0)),
          pl.BlockSpec((BM, 1), lambda i: (i, 0)),
          pl.BlockSpec((1, D_HID), lambda i: (0, 0)),
          pl.BlockSpec((D_HID, D_HID2), lambda i: (0, 0)),
      ],
      out_specs=pl.BlockSpec((BM, D_HID2), lambda i: (i, 0)),
      out_shape=jax.ShapeDtypeStruct((N_NODES, D_HID2), jnp.float32),
  )(alo, ahi, hlo, hhi, dinv, b1, w2)


def _tc3(agg, h2p, dinv, b2, wfct, bfc2):
  grid = (N_NODES // BM,)
  return pl.pallas_call(
      _tc3_body,
      grid=grid,
      in_specs=[
          pl.BlockSpec((NC, BM, D_HID2), lambda i: (0, i, 0)),
          pl.BlockSpec((BM, D_HID2), lambda i: (i, 0)),
          pl.BlockSpec((BM, 1), lambda i: (i, 0)),
          pl.BlockSpec((1, D_HID2), lambda i: (0, 0)),
          pl.BlockSpec((2, D_HID2), lambda i: (0, 0)),
          pl.BlockSpec((1, 2), lambda i: (0, 0)),
      ],
      out_specs=pl.BlockSpec((BM, 2), lambda i: (i, 0)),
      out_shape=jax.ShapeDtypeStruct((N_NODES, 2), jnp.float32),
  )(agg, h2p, dinv, b2, wfct, bfc2)


def kernel(x, edge_index, W1, b1, W2, b2, Wfc, bfc):
  ei = edge_index.astype(jnp.int32)
  src = ei[0].reshape(NW, NCHUNK, CH)
  dst = ei[1].reshape(NW, NCHUNK, CH)

  zeros_n1 = jnp.zeros((N_NODES, 1), jnp.float32)
  ones_ch1 = jnp.ones((CH, 1), jnp.float32)
  deg2 = _deg_kernel()(dst, zeros_n1, ones_ch1)

  hlo, hhi, dinv = _tc1(x, W1, deg2[0], deg2[1])
  alo, ahi = _make_agg_kernel(2)(hlo, hhi, src, dst)
  h2p = _tc2(alo, ahi, hlo, hhi, dinv, b1.reshape(1, D_HID), W2)
  (agg2,) = _make_agg_kernel(1)(h2p, src, dst)
  out = _tc3(agg2, h2p, dinv, b2.reshape(1, D_HID2), Wfc.T,
             bfc.reshape(1, 2))
  return out


# speedup vs baseline: 1.4312x; 1.0135x over previous
import jax
import jax.numpy as jnp
from jax import lax
from jax.experimental import pallas as pl
from jax.experimental.pallas import tpu as pltpu
from jax.experimental.pallas import tpu_sc as plsc

N_NODES = 10000
N_EDGES = 320000
D_IN = 128
D_HID = 128
D_HID2 = 64
DH = 64

NC = 2
NS = 16
NW = NC * NS

EPW = N_EDGES // NW
CH = 80
NCHUNK = EPW // CH
RPS = N_NODES // NS
ZR = 125


def _zero_fill(zbuf, rows, width):
  z16 = jnp.zeros((16,), jnp.float32)

  def body(i, carry):
    for j in range(width // 16):
      zbuf[i, pl.ds(j * 16, 16)] = z16
    return carry

  lax.fori_loop(0, rows, body, 0)


def _writeback(acc_sh, out_hbm, c, s):
  @pl.when(s < 10)
  def _():
    pltpu.sync_copy(acc_sh.at[pl.ds(s * 1000, 1000)],
                    out_hbm.at[c, pl.ds(s * 1000, 1000)])


def _deg_body(dst_hbm, zeros_hbm, ones_hbm, out_hbm, dbuf, ones_v, acc_sh,
              dsem):
  c = lax.axis_index("c")
  s = lax.axis_index("s")
  wid = s * NC + c

  pltpu.sync_copy(ones_hbm, ones_v)
  @pl.when(s < 10)
  def _():
    pltpu.sync_copy(zeros_hbm.at[pl.ds(s * 1000, 1000)],
                    acc_sh.at[pl.ds(s * 1000, 1000)])
  plsc.subcore_barrier()

  pltpu.sync_copy(dst_hbm.at[wid], dbuf)

  def chunk(i, carry):
    pltpu.sync_copy(ones_v, acc_sh.at[dbuf.at[i]], add=True)
    return carry

  lax.fori_loop(0, NCHUNK, chunk, 0)
  plsc.subcore_barrier()
  _writeback(acc_sh, out_hbm, c, s)


def _deg_kernel():
  mesh = plsc.VectorSubcoreMesh(core_axis_name="c", subcore_axis_name="s")
  return pl.kernel(
      _deg_body,
      compiler_params=pltpu.CompilerParams(use_tc_tiling_on_sc=False),
      out_type=jax.ShapeDtypeStruct((NC, N_NODES, 1), jnp.float32),
      mesh=mesh,
      scratch_types=[
          pltpu.VMEM((NCHUNK, CH), jnp.int32),
          pltpu.VMEM((CH, 1), jnp.float32),
          pltpu.VMEM_SHARED((N_NODES, 1), jnp.float32),
          pltpu.SemaphoreType.DMA,
      ],
  )


def _agg_pass(h_hbm, out_hbm, sbuf, dbuf, rows, zbuf, acc_sh, gs, ss, c, s):
  for k in range(RPS // ZR):
    pltpu.sync_copy(zbuf, acc_sh.at[pl.ds(s * RPS + k * ZR, ZR)])
  plsc.subcore_barrier()

  def g_start(r, i):
    pltpu.async_copy(h_hbm.at[sbuf.at[i]], rows[r], gs[r])

  def g_wait(r):
    pltpu.make_async_copy(h_hbm.at[pl.ds(0, CH)], rows[r], gs[r]).wait()

  def s_start(r, i):
    pltpu.async_copy(rows[r], acc_sh.at[dbuf.at[i]], ss[r], add=True)

  def s_wait(r):
    pltpu.make_async_copy(rows[r], acc_sh.at[pl.ds(0, CH)], ss[r]).wait()

  def step(r, i):
    g_wait(r)
    s_start(r, i)
    s_wait((r + 2) % 3)
    g_start((r + 2) % 3, i + 2)

  g_start(0, 0)
  g_start(1, 1)
  pltpu.async_copy(zbuf.at[pl.ds(0, CH)], acc_sh.at[dbuf.at[0]], ss[2],
                   add=True)

  def body(k, carry):
    i = 3 * k
    step(0, i)
    step(1, i + 1)
    step(2, i + 2)
    return carry

  lax.fori_loop(0, (NCHUNK - 2) // 3, body, 0)
  g_wait(0)
  s_start(0, NCHUNK - 2)
  s_wait(2)
  g_wait(1)
  s_start(1, NCHUNK - 1)
  s_wait(0)
  s_wait(1)
  plsc.subcore_barrier()
  _writeback(acc_sh, out_hbm, c, s)
  plsc.subcore_barrier()


def _make_agg_kernel(nhalves):
  def body(*refs):
    h_hbms = refs[:nhalves]
    src_hbm = refs[nhalves]
    dst_hbm = refs[nhalves + 1]
    out_hbms = refs[nhalves + 2:2 * nhalves + 2]
    (sbuf, dbuf, rows_0, rows_1, rows_2, zbuf, acc_sh,
     gs0, gs1, gs2, ss0, ss1, ss2) = refs[2 * nhalves + 2:]

    c = lax.axis_index("c")
    s = lax.axis_index("s")
    wid = s * NC + c

    _zero_fill(zbuf, ZR, DH)
    pltpu.sync_copy(src_hbm.at[wid], sbuf)
    pltpu.sync_copy(dst_hbm.at[wid], dbuf)

    for h_hbm, out_hbm in zip(h_hbms, out_hbms):
      _agg_pass(h_hbm, out_hbm, sbuf, dbuf, (rows_0, rows_1, rows_2), zbuf,
                acc_sh, (gs0, gs1, gs2), (ss0, ss1, ss2), c, s)

  mesh = plsc.VectorSubcoreMesh(core_axis_name="c", subcore_axis_name="s")
  return pl.kernel(
      body,
      compiler_params=pltpu.CompilerParams(use_tc_tiling_on_sc=False),
      out_type=[jax.ShapeDtypeStruct((NC, N_NODES, DH), jnp.float32)
                for _ in range(nhalves)],
      mesh=mesh,
      scratch_types=[
          pltpu.VMEM((NCHUNK, CH), jnp.int32),
          pltpu.VMEM((NCHUNK, CH), jnp.int32),
          pltpu.VMEM((CH, DH), jnp.float32),
          pltpu.VMEM((CH, DH), jnp.float32),
          pltpu.VMEM((CH, DH), jnp.float32),
          pltpu.VMEM((ZR, DH), jnp.float32),
          pltpu.VMEM_SHARED((N_NODES, DH), jnp.float32),
          pltpu.SemaphoreType.DMA,
          pltpu.SemaphoreType.DMA,
          pltpu.SemaphoreType.DMA,
          pltpu.SemaphoreType.DMA,
          pltpu.SemaphoreType.DMA,
          pltpu.SemaphoreType.DMA,
      ],
  )


BM = 2000


def _tc1_body(x_ref, w_ref, da_ref, db_ref, hlo_ref, hhi_ref, dinv_ref):
  deg = da_ref[...] + db_ref[...] + 1.0
  dinv = lax.rsqrt(deg)
  h = dinv * jnp.dot(x_ref[...], w_ref[...],
                     preferred_element_type=jnp.float32)
  hlo_ref[...] = h[:, :DH]
  hhi_ref[...] = h[:, DH:]
  dinv_ref[...] = dinv


def _tc2_body(alo_ref, ahi_ref, hlo_ref, hhi_ref, dinv_ref, b1_ref, w2_ref,
              out_ref):
  dinv = dinv_ref[...]
  zlo = (alo_ref[0] + alo_ref[1] + hlo_ref[...]) * dinv + b1_ref[:, :DH]
  zhi = (ahi_ref[0] + ahi_ref[1] + hhi_ref[...]) * dinv + b1_ref[:, DH:]
  r = jnp.concatenate([jnp.maximum(zlo, 0.0), jnp.maximum(zhi, 0.0)], axis=1)
  out_ref[...] = dinv * jnp.dot(r, w2_ref[...],
                                preferred_element_type=jnp.float32)


def _tc3_body(agg_ref, h2p_ref, dinv_ref, b2_ref, wfct_ref, bfc_ref, out_ref):
  dinv = dinv_ref[...]
  z = (agg_ref[0] + agg_ref[1] + h2p_ref[...]) * dinv + b2_ref[...]
  r = jnp.maximum(z, 0.0)
  l0 = jnp.sum(r * wfct_ref[0:1, :], axis=1, keepdims=True) + bfc_ref[:, 0:1]
  l1 = jnp.sum(r * wfct_ref[1:2, :], axis=1, keepdims=True) + bfc_ref[:, 1:2]
  m = jnp.maximum(l0, l1)
  lse = m + jnp.log(jnp.exp(l0 - m) + jnp.exp(l1 - m))
  out_ref[...] = jnp.concatenate([l0 - lse, l1 - lse], axis=1)


def _tc1(x, w1, dega, degb):
  grid = (N_NODES // BM,)
  return pl.pallas_call(
      _tc1_body,
      grid=grid,
      in_specs=[
          pl.BlockSpec((BM, D_IN), lambda i: (i, 0)),
          pl.BlockSpec((D_IN, D_HID), lambda i: (0, 0)),
          pl.BlockSpec((BM, 1), lambda i: (i, 0)),
          pl.BlockSpec((BM, 1), lambda i: (i, 0)),
      ],
      out_specs=[
          pl.BlockSpec((BM, DH), lambda i: (i, 0)),
          pl.BlockSpec((BM, DH), lambda i: (i, 0)),
          pl.BlockSpec((BM, 1), lambda i: (i, 0)),
      ],
      out_shape=[
          jax.ShapeDtypeStruct((N_NODES, DH), jnp.float32),
          jax.ShapeDtypeStruct((N_NODES, DH), jnp.float32),
          jax.ShapeDtypeStruct((N_NODES, 1), jnp.float32),
      ],
  )(x, w1, dega, degb)


def _tc2(alo, ahi, hlo, hhi, dinv, b1, w2):
  grid = (N_NODES // BM,)
  return pl.pallas_call(
      _tc2_body,
      grid=grid,
      in_specs=[
          pl.BlockSpec((NC, BM, DH), lambda i: (0, i, 0)),
          pl.BlockSpec((NC, BM, DH), lambda i: (0, i, 0)),
          pl.BlockSpec((BM, DH), lambda i: (i, 0)),
          pl.BlockSpec((BM, DH), lambda i: (i, 0)),
          pl.BlockSpec((BM, 1), lambda i: (i, 0)),
          pl.BlockSpec((1, D_HID), lambda i: (0, 0)),
          pl.BlockSpec((D_HID, D_HID2), lambda i: (0, 0)),
      ],
      out_specs=pl.BlockSpec((BM, D_HID2), lambda i: (i, 0)),
      out_shape=jax.ShapeDtypeStruct((N_NODES, D_HID2), jnp.float32),
  )(alo, ahi, hlo, hhi, dinv, b1, w2)


def _tc3(agg, h2p, dinv, b2, wfct, bfc2):
  grid = (N_NODES // BM,)
  return pl.pallas_call(
      _tc3_body,
      grid=grid,
      in_specs=[
          pl.BlockSpec((NC, BM, D_HID2), lambda i: (0, i, 0)),
          pl.BlockSpec((BM, D_HID2), lambda i: (i, 0)),
          pl.BlockSpec((BM, 1), lambda i: (i, 0)),
          pl.BlockSpec((1, D_HID2), lambda i: (0, 0)),
          pl.BlockSpec((2, D_HID2), lambda i: (0, 0)),
          pl.BlockSpec((1, 2), lambda i: (0, 0)),
      ],
      out_specs=pl.BlockSpec((BM, 2), lambda i: (i, 0)),
      out_shape=jax.ShapeDtypeStruct((N_NODES, 2), jnp.float32),
  )(agg, h2p, dinv, b2, wfct, bfc2)


def kernel(x, edge_index, W1, b1, W2, b2, Wfc, bfc):
  ei = edge_index.astype(jnp.int32)
  src = ei[0].reshape(NW, NCHUNK, CH)
  dst = ei[1].reshape(NW, NCHUNK, CH)

  zeros_n1 = jnp.zeros((N_NODES, 1), jnp.float32)
  ones_ch1 = jnp.ones((CH, 1), jnp.float32)
  deg2 = _deg_kernel()(dst, zeros_n1, ones_ch1)

  hlo, hhi, dinv = _tc1(x, W1, deg2[0], deg2[1])
  alo, ahi = _make_agg_kernel(2)(hlo, hhi, src, dst)
  h2p = _tc2(alo, ahi, hlo, hhi, dinv, b1.reshape(1, D_HID), W2)
  (agg2,) = _make_agg_kernel(1)(h2p, src, dst)
  out = _tc3(agg2, h2p, dinv, b2.reshape(1, D_HID2), Wfc.T,
             bfc.reshape(1, 2))
  return out
